# baseline (speedup 1.0000x reference)
import jax
import jax.numpy as jnp
from jax import lax
from jax.experimental import pallas as pl
from jax.experimental.pallas import tpu as pltpu

NSEG = 16
BLK = 4096


def _treemax(t):
    r = t.shape[0]
    while r > 8:
        r //= 2
        t = jnp.maximum(t[:r], t[r:])
    return jnp.max(t, axis=0, keepdims=True)


def _pool_kernel(blo_ref, bhi_ref, cum_ref, x_ref, pos_ref, w1_ref, w2_ref,
                 b_ref, out_ref):
    i = pl.program_id(0)
    nblk = pl.num_programs(0)
    base = i * BLK

    @pl.when(i == 0)
    def _init():
        out_ref[...] = jnp.full((NSEG, 64), -jnp.inf, dtype=jnp.float32)

    z = jnp.dot(x_ref[...], w1_ref[...], preferred_element_type=jnp.float32)
    z = z + jnp.dot(pos_ref[...], w2_ref[...], preferred_element_type=jnp.float32)

    lo = blo_ref[i]
    hi = bhi_ref[i]
    riota = lax.broadcasted_iota(jnp.int32, (BLK, 1), 0)

    end_lo = cum_ref[lo + 1] - base
    start_hi = cum_ref[hi] - base
    zz = jnp.concatenate(
        [jnp.where(riota < end_lo, z, -jnp.inf),
         jnp.where(riota >= start_hi, z, -jnp.inf)], axis=1)
    v2 = _treemax(zz)
    cur = out_ref[pl.ds(lo, 1), :]
    out_ref[pl.ds(lo, 1), :] = jnp.maximum(cur, v2[:, :64])
    cur = out_ref[pl.ds(hi, 1), :]
    out_ref[pl.ds(hi, 1), :] = jnp.maximum(cur, v2[:, 64:])

    @pl.when(hi > lo + 1)
    def _interior():
        def body(s, _):
            start = cum_ref[s] - base
            end = cum_ref[s + 1] - base
            m = jnp.logical_and(riota >= start, riota < end)
            v = _treemax(jnp.where(m, z, -jnp.inf))
            cur2 = out_ref[pl.ds(s, 1), :]
            out_ref[pl.ds(s, 1), :] = jnp.maximum(cur2, v)
            return 0

        lax.fori_loop(lo + 1, hi, body, 0)

    @pl.when(i == nblk - 1)
    def _finish():
        acc = out_ref[...]
        res = jnp.maximum(acc + b_ref[...], 0.0)
        out_ref[...] = jnp.where(acc == -jnp.inf, acc, res)


def kernel(x, pos, W, b, batch):
    n = x.shape[0]
    nblk = n // BLK

    w1 = W[:, :61].T
    w2 = W[:, 61:].T
    b2 = b.reshape(1, 64)
    batch = batch.astype(jnp.int32)
    blo = batch[::BLK]
    bhi = batch[BLK - 1::BLK]
    cum = jnp.searchsorted(batch, jnp.arange(NSEG + 1, dtype=jnp.int32)
                           ).astype(jnp.int32)

    grid_spec = pltpu.PrefetchScalarGridSpec(
        num_scalar_prefetch=3,
        grid=(nblk,),
        in_specs=[
            pl.BlockSpec((BLK, 61), lambda i, *_: (i, 0)),
            pl.BlockSpec((BLK, 3), lambda i, *_: (i, 0)),
            pl.BlockSpec((61, 64), lambda i, *_: (0, 0)),
            pl.BlockSpec((3, 64), lambda i, *_: (0, 0)),
            pl.BlockSpec((1, 64), lambda i, *_: (0, 0)),
        ],
        out_specs=pl.BlockSpec((NSEG, 64), lambda i, *_: (0, 0)),
    )

    return pl.pallas_call(
        _pool_kernel,
        grid_spec=grid_spec,
        out_shape=jax.ShapeDtypeStruct((NSEG, 64), jnp.float32),
    )(blo, bhi, cum, x, pos, w1, w2, b2)
